# Initial kernel scaffold; baseline (speedup 1.0000x reference)
import jax
import jax.numpy as jnp
from jax.experimental import pallas as pl


def kernel(loss, dummy):
    raise NotImplementedError("write your pallas kernel here")


# speedup vs baseline: 15.5340x; 15.5340x over previous
import functools
import jax
import jax.numpy as jnp
from jax.experimental import pallas as pl
from jax.experimental.pallas import tpu as pltpu

_ROWS = 16


def _topk_mean_block(loss_ref, out_ref, *, k):
    i = pl.program_id(0)
    x = loss_ref[...]
    bits = jax.lax.bitcast_convert_type(x, jnp.uint32)
    sign = bits >> 31
    u = bits ^ ((jnp.uint32(0) - sign) | jnp.uint32(0x80000000))
    r = x.shape[0]
    kf = jnp.float32(k)

    def body(_, carry):
        lo, hi = carry
        mid = lo + ((hi - lo) >> 1)
        cnt = jnp.sum((u >= mid).astype(jnp.float32), axis=1, keepdims=True)
        take = cnt >= kf
        return jnp.where(take, mid, lo), jnp.where(take, hi, mid)

    lo0 = jnp.zeros((r, 1), jnp.uint32)
    hi0 = jnp.full((r, 1), jnp.uint32(0xFFFFFFFF))
    t, _ = jax.lax.fori_loop(0, 32, body, (lo0, hi0))
    gt = u > t
    cnt_gt = jnp.sum(gt.astype(jnp.float32), axis=1, keepdims=True)
    sum_gt = jnp.sum(jnp.where(gt, x, 0.0), axis=1, keepdims=True)
    tval = jnp.max(jnp.where(u == t, x, -jnp.inf), axis=1, keepdims=True)
    row_sum = sum_gt + (kf - cnt_gt) * tval
    block = jnp.sum(row_sum)

    @pl.when(i == 0)
    def _():
        out_ref[0, 0] = 0.0

    out_ref[0, 0] += block


@jax.jit
def kernel(loss, dummy):
    b = loss.shape[0]
    loss = loss.reshape(b, -1)
    p = loss.shape[1]
    k = int(0.25 * p)
    grid = b // _ROWS
    out = pl.pallas_call(
        functools.partial(_topk_mean_block, k=k),
        grid=(grid,),
        in_specs=[pl.BlockSpec((_ROWS, p), lambda i: (i, 0))],
        out_specs=pl.BlockSpec((1, 1), lambda i: (0, 0), memory_space=pltpu.SMEM),
        out_shape=jax.ShapeDtypeStruct((1, 1), jnp.float32),
    )(loss)
    return out[0, 0] / (b * k)
